# baseline (speedup 1.0000x reference)
import functools

import jax
import jax.numpy as jnp
from jax import lax
from jax.experimental import pallas as pl
from jax.experimental.pallas import tpu as pltpu
from jax.experimental.pallas import tpu_sc as plsc

BATCH = 16384
SEQ = 50
DIM = 64
NUM_CORES = 2
NUM_SUBCORES = 16
NW = NUM_CORES * NUM_SUBCORES
B_PER_W = BATCH // NW
G = 8
T = B_PER_W // G

_mesh = plsc.VectorSubcoreMesh(core_axis_name="c", subcore_axis_name="s")


@functools.partial(
    pl.kernel,
    mesh=_mesh,
    out_type=jax.ShapeDtypeStruct((BATCH, SEQ, DIM), jnp.float32),
    scratch_types=[
        pltpu.VMEM((B_PER_W, SEQ), jnp.int32),
        pltpu.VMEM((2, G, SEQ, DIM), jnp.float32),
        pltpu.SemaphoreType.DMA,
        pltpu.SemaphoreType.DMA,
        pltpu.SemaphoreType.DMA,
        pltpu.SemaphoreType.DMA,
    ],
    compiler_params=pltpu.CompilerParams(use_tc_tiling_on_sc=False),
)
def _gather_kernel(table_hbm, idx_hbm, out_hbm, idx_v, bufs, g0, g1, s0, s1):
    wid = lax.axis_index("s") * NUM_CORES + lax.axis_index("c")
    base = wid * B_PER_W
    pltpu.sync_copy(idx_hbm.at[pl.ds(base, B_PER_W)], idx_v)

    gsems = (g0, g1)
    ssems = (s0, s1)

    def fire_gathers(c, t):
        for i in range(G):
            pltpu.async_copy(
                table_hbm.at[idx_v.at[t * G + i]], bufs.at[c, i], gsems[c]
            )

    def wait_gathers(c):
        pltpu.make_async_copy(
            out_hbm.at[pl.ds(0, G)], bufs.at[c], gsems[c]
        ).wait()

    def fire_stores(c, t):
        pltpu.async_copy(
            bufs.at[c], out_hbm.at[pl.ds(base + t * G, G)], ssems[c]
        )

    def wait_stores(c):
        pltpu.make_async_copy(
            bufs.at[c], out_hbm.at[pl.ds(0, G)], ssems[c]
        ).wait()

    fire_gathers(0, 0)
    wait_gathers(0)
    fire_gathers(1, 1)
    fire_stores(0, 0)

    def body(t2, carry):
        t1 = 2 * t2 + 1
        wait_gathers(1)
        wait_stores(0)
        fire_gathers(0, t1 + 1)
        fire_stores(1, t1)
        t0 = t1 + 1
        wait_gathers(0)
        wait_stores(1)
        fire_gathers(1, t0 + 1)
        fire_stores(0, t0)
        return carry

    lax.fori_loop(0, (T - 2) // 2, body, 0)

    wait_gathers(1)
    wait_stores(0)
    fire_stores(1, T - 1)
    wait_stores(1)


def kernel(x, weight):
    return _gather_kernel(weight, x.astype(jnp.int32))


# speedup vs baseline: 1.3903x; 1.3903x over previous
import functools

import jax
import jax.numpy as jnp
from jax import lax
from jax.experimental import pallas as pl
from jax.experimental.pallas import tpu as pltpu
from jax.experimental.pallas import tpu_sc as plsc

BATCH = 16384
SEQ = 50
SEQ_PAD = 56
DIM = 64
VOCAB = 1_000_000
NUM_CORES = 2
NUM_SUBCORES = 16
D_PER_CORE = DIM // NUM_CORES
B_PER_SUB = BATCH // NUM_SUBCORES
GCHUNK = 128
SLAB = 8

_mesh = plsc.VectorSubcoreMesh(core_axis_name="c", subcore_axis_name="s")


@functools.partial(
    pl.kernel,
    mesh=_mesh,
    out_type=jax.ShapeDtypeStruct((SEQ, DIM, BATCH), jnp.float32),
    scratch_types=[
        pltpu.VMEM_SHARED((VOCAB,), jnp.float32),
        pltpu.VMEM((SEQ_PAD, B_PER_SUB), jnp.int32),
        pltpu.VMEM((SLAB, 1, B_PER_SUB), jnp.float32),
        pltpu.SemaphoreType.DMA,
    ],
)
def _gather_kernel(table_hbm, idx_hbm, out_hbm, row_sp, idx_v, gbuf, gsem):
    c = lax.axis_index("c")
    t = lax.axis_index("s")
    b0 = t * B_PER_SUB
    pltpu.sync_copy(idx_hbm.at[:, pl.ds(b0, B_PER_SUB)], idx_v)

    def gather_seq(s, r):
        for k in range(B_PER_SUB // GCHUNK):
            pltpu.async_copy(
                row_sp.at[idx_v.at[s, pl.ds(k * GCHUNK, GCHUNK)]],
                gbuf.at[r, 0, pl.ds(k * GCHUNK, GCHUNK)],
                gsem,
            )
        pltpu.make_async_copy(
            table_hbm.at[0, pl.ds(0, B_PER_SUB)], gbuf.at[r, 0], gsem
        ).wait()

    def per_dim(dd, carry):
        d = c * D_PER_CORE + dd
        plsc.subcore_barrier()

        @pl.when(t == 0)
        def _load_row():
            pltpu.sync_copy(table_hbm.at[d], row_sp)

        plsc.subcore_barrier()

        def per_slab(so, carry2):
            def inner(s8, carry3):
                gather_seq(so * SLAB + s8, s8)
                return carry3

            lax.fori_loop(0, SLAB, inner, 0)
            pltpu.sync_copy(
                gbuf,
                out_hbm.at[
                    pl.ds(so * SLAB, SLAB), pl.ds(d, 1), pl.ds(b0, B_PER_SUB)
                ],
            )
            return carry2

        lax.fori_loop(0, SEQ // SLAB, per_slab, 0)
        gather_seq(48, 0)
        gather_seq(49, 1)
        pltpu.sync_copy(
            gbuf.at[pl.ds(0, 2)],
            out_hbm.at[pl.ds(48, 2), pl.ds(d, 1), pl.ds(b0, B_PER_SUB)],
        )
        return carry

    lax.fori_loop(0, D_PER_CORE, per_dim, 0)


def kernel(x, weight):
    x_t = jnp.pad(x.astype(jnp.int32).T, ((0, SEQ_PAD - SEQ), (0, 0)))
    out_t = _gather_kernel(weight.T, x_t)
    return out_t.transpose(2, 0, 1)


# speedup vs baseline: 1.6724x; 1.2030x over previous
import functools

import jax
import jax.numpy as jnp
from jax import lax
from jax.experimental import pallas as pl
from jax.experimental.pallas import tpu as pltpu
from jax.experimental.pallas import tpu_sc as plsc

BATCH = 16384
SEQ = 50
DIM = 64
VOCAB = 1_000_000
NUM_CORES = 2
NUM_SUBCORES = 16
D_PER_CORE = DIM // NUM_CORES
B_PER_SUB = BATCH // NUM_SUBCORES
SLAB = 4
N_SLABS = 12

_mesh = plsc.VectorSubcoreMesh(core_axis_name="c", subcore_axis_name="s")


@functools.partial(
    pl.kernel,
    mesh=_mesh,
    out_type=jax.ShapeDtypeStruct((SEQ, DIM, BATCH), jnp.float32),
    scratch_types=[
        pltpu.VMEM_SHARED((VOCAB,), jnp.float32),
        pltpu.VMEM((SEQ * B_PER_SUB,), jnp.int32),
        pltpu.VMEM((2, SLAB, 1, B_PER_SUB), jnp.float32),
        pltpu.SemaphoreType.DMA,
        pltpu.SemaphoreType.DMA,
        pltpu.SemaphoreType.DMA,
    ],
)
def _gather_kernel(table_hbm, idx_hbm, out_hbm, row_sp, idx_v, gbuf, gsem, s0, s1):
    c = lax.axis_index("c")
    t = lax.axis_index("s")
    b0 = t * B_PER_SUB
    ssems = (s0, s1)

    for si in range(SEQ):
        pltpu.async_copy(
            idx_hbm.at[si].at[pl.ds(b0, B_PER_SUB)],
            idx_v.at[pl.ds(si * B_PER_SUB, B_PER_SUB)],
            gsem,
        )
    for si in range(SEQ):
        pltpu.make_async_copy(
            idx_hbm.at[0].at[pl.ds(0, B_PER_SUB)],
            idx_v.at[pl.ds(0, B_PER_SUB)],
            gsem,
        ).wait()

    def fire_gather(s, bsel, r):
        pltpu.async_copy(
            row_sp.at[idx_v.at[pl.ds(s * B_PER_SUB, B_PER_SUB)]],
            gbuf.at[bsel, r, 0],
            gsem,
        )

    def drain_gather():
        pltpu.make_async_copy(
            table_hbm.at[0].at[pl.ds(0, B_PER_SUB)], gbuf.at[0, 0, 0], gsem
        ).wait()

    def store_slab(so, bsel, d, n):
        pltpu.async_copy(
            gbuf.at[bsel, pl.ds(0, n)],
            out_hbm.at[pl.ds(so * SLAB, n), pl.ds(d, 1), pl.ds(b0, B_PER_SUB)],
            ssems[bsel],
        )

    def drain_store(bsel, n):
        pltpu.make_async_copy(
            gbuf.at[bsel, pl.ds(0, n)],
            out_hbm.at[pl.ds(0, n), pl.ds(0, 1), pl.ds(0, B_PER_SUB)],
            ssems[bsel],
        ).wait()

    def gather_slab(so, bsel, d):
        def inner(s4, carry3):
            fire_gather(so * SLAB + s4, bsel, s4)

            @pl.when(s4 > 0)
            def _lagged():
                drain_gather()

            return carry3

        lax.fori_loop(0, SLAB, inner, 0)
        drain_gather()
        store_slab(so, bsel, d, SLAB)

    def per_dim(dd, carry):
        d = c * D_PER_CORE + dd
        plsc.subcore_barrier()

        @pl.when(t == 0)
        def _load_row():
            pltpu.sync_copy(table_hbm.at[d], row_sp)

        plsc.subcore_barrier()

        gather_slab(0, 0, d)
        gather_slab(1, 1, d)

        def per_pair(p, carry2):
            so = 2 * p + 2
            drain_store(0, SLAB)
            gather_slab(so, 0, d)
            drain_store(1, SLAB)
            gather_slab(so + 1, 1, d)
            return carry2

        lax.fori_loop(0, (N_SLABS - 2) // 2, per_pair, 0)

        drain_store(0, SLAB)
        fire_gather(48, 0, 0)
        fire_gather(49, 0, 1)
        drain_gather()
        drain_gather()
        pltpu.sync_copy(
            gbuf.at[0, pl.ds(0, 2)],
            out_hbm.at[pl.ds(48, 2), pl.ds(d, 1), pl.ds(b0, B_PER_SUB)],
        )
        drain_store(1, SLAB)
        return carry

    lax.fori_loop(0, D_PER_CORE, per_dim, 0)


def kernel(x, weight):
    out_t = _gather_kernel(weight.T, x.astype(jnp.int32).T)
    return out_t.transpose(2, 0, 1)
